# baseline (speedup 1.0000x reference)
import functools

import jax
import jax.numpy as jnp
from jax import lax
from jax.experimental import pallas as pl
from jax.experimental.pallas import tpu as pltpu
from jax.experimental.pallas import tpu_sc as plsc

NC = 2
NS = 16
L = 16
NW = NC * NS

B = 16384
EMB = 64
N = 1000000
WSPAN = 32768
CW = 512
NCH = WSPAN // CW
TAIL = (N // 128) * 128
OCAP = 1216
RCAP = 256
TRASH = B

BW = B // NW
CHUNK = 128
NCHUNK = BW // CHUNK


def _sel_pass(ids_hbm, idbuf_v, owned_v, wstart, t_unused, cur0):
    iota16 = lax.iota(jnp.int32, L)
    cur = cur0
    for cc in range(B // 4096):
        pltpu.sync_copy(ids_hbm.at[pl.ds(cc * 4096, 4096)], idbuf_v)

        def sel(k, cur):
            v = idbuf_v[pl.ds(k * L, L)]
            rel = v - wstart
            m = (rel >= 0) & (rel < WSPAN)
            b_vec = cc * 4096 + k * L + iota16
            packed = lax.shift_left(rel, 14) | b_vec
            plsc.store_compressed(owned_v.at[pl.ds(cur, L)], packed, mask=m)
            n = plsc.all_reduce_population_count(m)[0]
            return jnp.minimum(cur + n, OCAP - L)

        cur = lax.fori_loop(0, 4096 // L, sel, cur)
    return cur


def _reorder(owned_v, ordered_v, smem_cur, tot):
    iota16 = lax.iota(jnp.int32, L)
    nvec = (tot + L - 1) // L
    cur2 = 0
    for c2 in range(NCH):
        smem_cur[c2] = cur2

        def scan(k, cur2):
            v = owned_v[pl.ds(k * L, L)]
            pos = k * L + iota16
            cid = lax.shift_right_logical(v, 14 + 9)
            m = (cid == c2) & (pos < tot)
            plsc.store_compressed(ordered_v.at[pl.ds(cur2, L)], v, mask=m)
            n = plsc.all_reduce_population_count(m)[0]
            return cur2 + n

        cur2 = lax.fori_loop(0, nvec, scan, cur2)
    smem_cur[NCH] = cur2


def _fill_trash(idxstage_v):
    trash = jnp.full((L,), TRASH, jnp.int32)
    for j in range(2):
        for k in range(128 // L):
            idxstage_v[j, pl.ds(k * L, L)] = trash


def _stream_pass(tab_hbm, stage_hbm, ordered_v, smem_cur, buf_v, tail_v,
                 rowstage_v, idxstage_v, sem0, sem1, semsc, wstart):
    iota16 = lax.iota(jnp.int32, L)
    dhis = []
    dlos = []
    for gg in range(4):
        d = 16 * gg + iota16
        dhis.append(lax.shift_right_logical(d, 3))
        dlos.append(lax.bitwise_and(d, jnp.full((L,), 7, jnp.int32)))
    lane0 = iota16 == 0

    def fire(c, parity):
        cs = wstart + c * CW
        sem = sem0 if parity == 0 else sem1

        @pl.when(cs + CW <= N)
        def _():
            pltpu.async_copy(tab_hbm.at[:, :, pl.ds(cs, CW)],
                             buf_v.at[parity], sem)

        @pl.when(cs == TAIL)
        def _():
            pltpu.async_copy(tab_hbm.at[:, :, pl.ds(TAIL, N - TAIL)],
                             tail_v, sem)

    def drain(c, parity):
        cs = wstart + c * CW
        sem = sem0 if parity == 0 else sem1

        @pl.when(cs + CW <= N)
        def _():
            pltpu.make_async_copy(tab_hbm.at[:, :, pl.ds(cs, CW)],
                                  buf_v.at[parity], sem).wait()

        @pl.when(cs == TAIL)
        def _():
            pltpu.make_async_copy(tab_hbm.at[:, :, pl.ds(TAIL, N - TAIL)],
                                  tail_v, sem).wait()

    fire(0, 0)

    def chunk_body(c, rcur):
        parity = lax.rem(c, 2)

        @pl.when((c + 1 < NCH) & (parity == 0))
        def _():
            fire(c + 1, 1)

        @pl.when((c + 1 < NCH) & (parity == 1))
        def _():
            fire(c + 1, 0)

        @pl.when(parity == 0)
        def _():
            drain(c, 0)

        @pl.when(parity == 1)
        def _():
            drain(c, 1)

        cs = wstart + c * CW
        is_tail = cs + CW > N
        cs_act = jnp.where(is_tail, TAIL, cs)
        s = smem_cur[c]
        e = smem_cur[c + 1]

        def extract(p, rcur):
            v = ordered_v[pl.ds(p, L)][0]
            relidx = lax.shift_right_logical(v, 14)
            b = lax.bitwise_and(v, 16383)
            col = wstart + relidx - cs_act
            colt = jnp.minimum(col, N - TAIL - 1)
            rr = jnp.minimum(rcur, RCAP - 1)
            for gg in range(4):
                vals_n = plsc.load_gather(
                    buf_v, [jnp.full((L,), parity, jnp.int32), dhis[gg],
                            dlos[gg], jnp.full((L,), col, jnp.int32)])
                vals_t = plsc.load_gather(
                    tail_v, [dhis[gg], dlos[gg],
                             jnp.full((L,), colt, jnp.int32)])
                vals = jnp.where(is_tail, vals_t, vals_n)
                rowstage_v[rr, pl.ds(gg * L, L)] = vals
            plsc.store_scatter(
                idxstage_v,
                [jnp.full((L,), lax.shift_right_logical(rr, 7), jnp.int32),
                 jnp.full((L,), lax.bitwise_and(rr, 127), jnp.int32)],
                jnp.full((L,), b, jnp.int32), mask=lane0)
            return rcur + 1

        rcur = lax.fori_loop(s, e, extract, rcur)

        @pl.when(lax.rem(c, 8) == 7)
        def _():
            for k in range(RCAP // 128):
                @pl.when(rcur > k * 128)
                def _():
                    cp = pltpu.async_copy(
                        rowstage_v.at[pl.ds(k * 128, 128), :],
                        stage_hbm.at[idxstage_v.at[k]], semsc)
                    cp.wait()
            _fill_trash(idxstage_v)

        return jnp.where(lax.rem(c, 8) == 7, 0, rcur)

    lax.fori_loop(0, NCH, chunk_body, 0)


def _phase_a_body(ut_hbm, it_hbm, u_id_hbm, i_id_hbm, ustage_hbm, istage_hbm,
                  idbuf_v, owned_u, owned_i, ordered_v, buf_v, tail_v,
                  rowstage_v, idxstage_v, sem0, sem1, semsc, smem_cur):
    wid = lax.axis_index("s") * NC + lax.axis_index("c")
    wstart = wid * WSPAN

    tot_u = _sel_pass(u_id_hbm, idbuf_v, owned_u, wstart, 0, 0)
    tot_i = _sel_pass(i_id_hbm, idbuf_v, owned_i, wstart, 1, 0)
    _fill_trash(idxstage_v)

    _reorder(owned_u, ordered_v, smem_cur, tot_u)
    _stream_pass(ut_hbm, ustage_hbm, ordered_v, smem_cur, buf_v, tail_v,
                 rowstage_v, idxstage_v, sem0, sem1, semsc, wstart)

    _reorder(owned_i, ordered_v, smem_cur, tot_i)
    _stream_pass(it_hbm, istage_hbm, ordered_v, smem_cur, buf_v, tail_v,
                 rowstage_v, idxstage_v, sem0, sem1, semsc, wstart)


def _phase_b_body(ustage_hbm, istage_hbm, u_id_hbm, i_id_hbm, ub_hbm, ib_hbm,
                  mean_hbm, out_hbm,
                  uidx_v, iidx_v, uhi_v, ihi_v, ulo_v, ilo_v,
                  urows_v, irows_v, ubias_v, ibias_v, out_v, mean_v, sem):
    wid = lax.axis_index("s") * NC + lax.axis_index("c")
    base = wid * BW

    for j in range(NCHUNK):
        pltpu.sync_copy(u_id_hbm.at[pl.ds(base + j * CHUNK, CHUNK)],
                        uidx_v.at[j])
        pltpu.sync_copy(i_id_hbm.at[pl.ds(base + j * CHUNK, CHUNK)],
                        iidx_v.at[j])
    pltpu.sync_copy(mean_hbm, mean_v.at[pl.ds(0, 1)])

    mask15 = jnp.full((L,), 15, jnp.int32)
    for j in range(NCHUNK):
        for k in range(CHUNK // L):
            sl = pl.ds(k * L, L)
            fl = pl.ds(j * CHUNK + k * L, L)
            uv = uidx_v[j, sl]
            iv = iidx_v[j, sl]
            uhi_v[j, sl] = lax.shift_right_logical(uv, 4)
            ihi_v[j, sl] = lax.shift_right_logical(iv, 4)
            ulo_v[fl] = lax.bitwise_and(uv, mask15)
            ilo_v[fl] = lax.bitwise_and(iv, mask15)

    copies = []
    for j in range(NCHUNK):
        sl = pl.ds(j * CHUNK, CHUNK)
        copies.append(pltpu.async_copy(ub_hbm.at[uhi_v.at[j]],
                                       ubias_v.at[sl], sem))
        copies.append(pltpu.async_copy(ib_hbm.at[ihi_v.at[j]],
                                       ibias_v.at[sl], sem))
    for c in copies:
        c.wait()

    iota16 = lax.iota(jnp.int32, L)
    mean_s = mean_v[pl.ds(0, L)][0]

    HB = BW // 2
    for h in range(2):
        pltpu.async_copy(ustage_hbm.at[pl.ds(base + h * HB, HB), :],
                         urows_v, sem).wait()
        pltpu.async_copy(istage_hbm.at[pl.ds(base + h * HB, HB), :],
                         irows_v, sem).wait()

        def group_body(g, carry):
            rows = g * L + iota16
            acc = jnp.zeros((L,), jnp.float32)

            def d_body(d, acc):
                dcol = jnp.full((L,), d, jnp.int32)
                u = plsc.load_gather(urows_v, [rows, dcol])
                iv = plsc.load_gather(irows_v, [rows, dcol])
                return acc + u * iv

            acc = lax.fori_loop(0, EMB, d_body, acc)
            sl = pl.ds(h * HB + g * L, L)
            ub = plsc.load_gather(ubias_v, [rows + h * HB, ulo_v[sl]])
            ib = plsc.load_gather(ibias_v, [rows + h * HB, ilo_v[sl]])
            out_v[sl] = acc + ub + ib + mean_s
            return carry

        lax.fori_loop(0, HB // L, group_body, 0)

    pltpu.sync_copy(out_v, out_hbm.at[pl.ds(base, BW)])


_MESH = dict(core_axis_name="c", subcore_axis_name="s",
             num_cores=NC, num_subcores=NS)


@jax.jit
def _mf(u_id, i_id, user_emb, user_bias, item_emb, item_bias, mean):
    ut3 = user_emb.T.reshape(8, 8, N)
    it3 = item_emb.T.reshape(8, 8, N)
    ub16 = user_bias.reshape(N // L, L)
    ib16 = item_bias.reshape(N // L, L)

    ustage, istage = pl.kernel(
        _phase_a_body,
        out_type=(jax.ShapeDtypeStruct((B + 1, 128), jnp.float32),
                  jax.ShapeDtypeStruct((B + 1, 128), jnp.float32)),
        mesh=plsc.VectorSubcoreMesh(**_MESH),
        scratch_types=[
            pltpu.VMEM((4096,), jnp.int32),
            pltpu.VMEM((OCAP,), jnp.int32),
            pltpu.VMEM((OCAP,), jnp.int32),
            pltpu.VMEM((OCAP,), jnp.int32),
            pltpu.VMEM((2, 8, 8, CW), jnp.float32),
            pltpu.VMEM((8, 8, N - TAIL), jnp.float32),
            pltpu.VMEM((RCAP, 128), jnp.float32),
            pltpu.VMEM((RCAP // 128, 128), jnp.int32),
            pltpu.SemaphoreType.DMA,
            pltpu.SemaphoreType.DMA,
            pltpu.SemaphoreType.DMA,
            pltpu.SMEM((NCH + 1,), jnp.int32),
        ],
        compiler_params=pltpu.CompilerParams(needs_layout_passes=False),
    )(ut3, it3, u_id, i_id)

    return pl.kernel(
        _phase_b_body,
        out_type=jax.ShapeDtypeStruct((B,), jnp.float32),
        mesh=plsc.VectorSubcoreMesh(**_MESH),
        scratch_types=[
            pltpu.VMEM((NCHUNK, CHUNK), jnp.int32),
            pltpu.VMEM((NCHUNK, CHUNK), jnp.int32),
            pltpu.VMEM((NCHUNK, CHUNK), jnp.int32),
            pltpu.VMEM((NCHUNK, CHUNK), jnp.int32),
            pltpu.VMEM((BW,), jnp.int32),
            pltpu.VMEM((BW,), jnp.int32),
            pltpu.VMEM((BW // 2, 128), jnp.float32),
            pltpu.VMEM((BW // 2, 128), jnp.float32),
            pltpu.VMEM((BW, L), jnp.float32),
            pltpu.VMEM((BW, L), jnp.float32),
            pltpu.VMEM((BW,), jnp.float32),
            pltpu.VMEM((L,), jnp.float32),
            pltpu.SemaphoreType.DMA,
        ],
        compiler_params=pltpu.CompilerParams(needs_layout_passes=False,
                                             use_tc_tiling_on_sc=False),
    )(ustage, istage, u_id, i_id, ub16, ib16, mean)


def kernel(u_id, i_id, user_emb, user_bias, item_emb, item_bias, mean):
    return _mf(u_id, i_id, user_emb, user_bias, item_emb, item_bias, mean)
